# Initial kernel scaffold; baseline (speedup 1.0000x reference)
import jax
import jax.numpy as jnp
from jax.experimental import pallas as pl


def kernel(x, tables):
    raise NotImplementedError("write your pallas kernel here")


# speedup vs baseline: 1.5141x; 1.5141x over previous
import functools

import jax
import jax.numpy as jnp
from jax import lax
from jax.experimental import pallas as pl
from jax.experimental.pallas import tpu as pltpu
from jax.experimental.pallas import tpu_sc as plsc

N_FIELDS = 26
VOCAB = 100000
D_EMB = 32
B = 1024
L = 20

NC = 2
NS = 16
LANES = 16
NW = NC * NS

N_TOTAL = B * L * N_FIELDS
PER_W = N_TOTAL // NW
BLK = 128
NBLK = PER_W // BLK


def _make_sc_gather():
    mesh = plsc.VectorSubcoreMesh(core_axis_name="c", subcore_axis_name="s")

    @functools.partial(
        pl.kernel,
        mesh=mesh,
        compiler_params=pltpu.CompilerParams(use_tc_tiling_on_sc=False),
        out_type=jax.ShapeDtypeStruct((N_TOTAL, D_EMB), jnp.float32),
        scratch_types=[
            pltpu.VMEM((BLK,), jnp.int32),
            pltpu.VMEM((BLK, D_EMB), jnp.float32),
            pltpu.SemaphoreType.DMA,
        ],
    )
    def k(x_hbm, tab_hbm, out_hbm, idx_v, rows_v, sem):
        wid = lax.axis_index("s") * NC + lax.axis_index("c")
        base = wid * PER_W

        def body(bi, carry):
            off = base + bi * BLK
            pltpu.sync_copy(x_hbm.at[pl.ds(off, BLK)], idx_v)

            def addoff(t, c):
                j = lax.iota(jnp.int32, LANES) + (off + t * LANES)
                f = lax.rem(j, N_FIELDS)
                idx_v[pl.ds(t * LANES, LANES)] = (
                    idx_v[pl.ds(t * LANES, LANES)] + f * VOCAB
                )
                return c

            lax.fori_loop(0, BLK // LANES, addoff, 0)

            pltpu.async_copy(tab_hbm.at[idx_v], rows_v, sem).wait()
            pltpu.sync_copy(rows_v, out_hbm.at[pl.ds(off, BLK)])
            return carry

        lax.fori_loop(0, NBLK, body, 0)

    return k


_sc_gather = _make_sc_gather()


def kernel(x, tables):
    x_flat = x.reshape(N_TOTAL).astype(jnp.int32)
    tab_flat = tables.reshape(N_FIELDS * VOCAB, D_EMB)
    out = _sc_gather(x_flat, tab_flat)
    return out.reshape(B, L, N_FIELDS, D_EMB)


# speedup vs baseline: 1.6605x; 1.0967x over previous
import functools

import jax
import jax.numpy as jnp
from jax import lax
from jax.experimental import pallas as pl
from jax.experimental.pallas import tpu as pltpu
from jax.experimental.pallas import tpu_sc as plsc

N_FIELDS = 26
VOCAB = 100000
D_EMB = 32
B = 1024
L = 20

NC = 2
NS = 16
LANES = 16
NW = NC * NS

N_TOTAL = B * L * N_FIELDS
PER_W = N_TOTAL // NW
SUB = 128
BLK = 1280
NSUB = BLK // SUB
NBLK = PER_W // BLK


def _make_sc_gather():
    mesh = plsc.VectorSubcoreMesh(core_axis_name="c", subcore_axis_name="s")

    @functools.partial(
        pl.kernel,
        mesh=mesh,
        compiler_params=pltpu.CompilerParams(use_tc_tiling_on_sc=False),
        out_type=jax.ShapeDtypeStruct((N_TOTAL, D_EMB), jnp.float32),
        scratch_types=[
            pltpu.VMEM((PER_W,), jnp.int32),
            pltpu.VMEM((BLK, D_EMB), jnp.float32),
            pltpu.VMEM((BLK, D_EMB), jnp.float32),
            pltpu.SemaphoreType.DMA,
            pltpu.SemaphoreType.DMA,
            pltpu.SemaphoreType.DMA,
            pltpu.SemaphoreType.DMA,
        ],
    )
    def k(x_hbm, tab_hbm, out_hbm, idx_v, rows0, rows1, sg0, sg1, sw0, sw1):
        wid = lax.axis_index("s") * NC + lax.axis_index("c")
        base = wid * PER_W

        pltpu.sync_copy(x_hbm.at[pl.ds(base, PER_W)], idx_v)

        def addoff(t, c):
            j = lax.iota(jnp.int32, LANES) + (base + t * LANES)
            idx_v[pl.ds(t * LANES, LANES)] = (
                idx_v[pl.ds(t * LANES, LANES)] + lax.rem(j, N_FIELDS) * VOCAB
            )
            return c

        lax.fori_loop(0, PER_W // LANES, addoff, 0)

        rows = [rows0, rows1]
        sg = [sg0, sg1]
        sw = [sw0, sw1]

        def issue_gather(g):
            b = g % 2
            return [
                pltpu.async_copy(
                    tab_hbm.at[idx_v.at[pl.ds(g * BLK + s * SUB, SUB)]],
                    rows[b].at[pl.ds(s * SUB, SUB)],
                    sg[b],
                )
                for s in range(NSUB)
            ]

        def issue_write(g):
            b = g % 2
            return pltpu.async_copy(
                rows[b], out_hbm.at[pl.ds(base + g * BLK, BLK)], sw[b]
            )

        gcopies = issue_gather(0)
        wcopies = [None] * NBLK
        for g in range(NBLK):
            if g + 1 < NBLK:
                if g - 1 >= 0:
                    wcopies[g - 1].wait()
                next_gcopies = issue_gather(g + 1)
            for c in gcopies:
                c.wait()
            wcopies[g] = issue_write(g)
            if g + 1 < NBLK:
                gcopies = next_gcopies
        wcopies[NBLK - 2].wait()
        wcopies[NBLK - 1].wait()

    return k


_sc_gather = _make_sc_gather()


def kernel(x, tables):
    x_flat = x.reshape(N_TOTAL).astype(jnp.int32)
    tab_flat = tables.reshape(N_FIELDS * VOCAB, D_EMB)
    out = _sc_gather(x_flat, tab_flat)
    return out.reshape(B, L, N_FIELDS, D_EMB)


# speedup vs baseline: 9.8872x; 5.9543x over previous
import functools

import jax
import jax.numpy as jnp
from jax import lax
from jax.experimental import pallas as pl
from jax.experimental.pallas import tpu as pltpu
from jax.experimental.pallas import tpu_sc as plsc

N_FIELDS = 26
VOCAB = 100000
D_EMB = 32
B = 1024
L = 20

NC = 2
NS = 16
LANES = 16
NW = NC * NS


def _make_sc_gather():
    mesh = plsc.VectorSubcoreMesh(core_axis_name="c", subcore_axis_name="s")

    @functools.partial(
        pl.kernel,
        mesh=mesh,
        compiler_params=pltpu.CompilerParams(
            use_tc_tiling_on_sc=True, needs_layout_passes=False
        ),
        out_type=jax.ShapeDtypeStruct((L, N_FIELDS, D_EMB, B), jnp.float32),
        scratch_types=[
            pltpu.VMEM((VOCAB,), jnp.float32),
            pltpu.VMEM((L, B), jnp.int32),
            pltpu.VMEM((B,), jnp.float32),
            pltpu.VMEM((B,), jnp.float32),
            pltpu.SemaphoreType.DMA,
            pltpu.SemaphoreType.DMA,
            pltpu.SemaphoreType.DMA,
            pltpu.SemaphoreType.DMA,
        ],
    )
    def k(x_hbm, tab_hbm, out_hbm, row_v, idx_v, outb0, outb1, sr, si, sw0, sw1):
        d = lax.axis_index("s") * NC + lax.axis_index("c")
        outb = [outb0, outb1]
        sw = [sw0, sw1]

        def per_field(f, c):
            cp_i = pltpu.async_copy(x_hbm.at[f], idx_v, si)
            cp_r = pltpu.async_copy(tab_hbm.at[f, d], row_v, sr)
            cp_i.wait()
            cp_r.wait()
            wc = [None, None]
            for l in range(L):
                bsel = l % 2
                if wc[bsel] is not None:
                    wc[bsel].wait()

                @plsc.parallel_loop(0, B // LANES, unroll=8)
                def gbody(g):
                    iv = idx_v[l, pl.ds(g * LANES, LANES)]
                    outb[bsel][pl.ds(g * LANES, LANES)] = plsc.load_gather(
                        row_v, [iv]
                    )

                wc[bsel] = pltpu.async_copy(
                    outb[bsel], out_hbm.at[l, f, d], sw[bsel]
                )
            wc[0].wait()
            wc[1].wait()
            return c

        lax.fori_loop(0, N_FIELDS, per_field, 0)

    return k


_sc_gather = _make_sc_gather()


def kernel(x, tables):
    x_t = x.astype(jnp.int32).transpose(2, 1, 0)
    tab_t = tables.transpose(0, 2, 1)
    out = _sc_gather(x_t, tab_t)
    return out.transpose(3, 0, 1, 2)
